# baseline (speedup 1.0000x reference)
import functools

import jax
import jax.numpy as jnp
from jax import lax
from jax.experimental import pallas as pl
from jax.experimental.pallas import tpu as pltpu
from jax.experimental.pallas import tpu_sc as plsc

_NUM_EMB = 1024
_DIM = 64
_B = 1024
_PAD = 128


def _argmin_body(w_ref, x_ref, idx_ref, wp_ref):
    wv = w_ref[...]
    xv = x_ref[...]
    wp_ref[:, :_DIM] = wv
    s = lax.dot_general(
        wv, xv, (((1,), (1,)), ((), ())),
        preferred_element_type=jnp.float32,
        precision=lax.Precision.HIGHEST,
    )
    wn = 0.5 * jnp.sum(wv * wv, axis=1)
    s = wn[:, None] - s
    m = jnp.min(s, axis=0, keepdims=True)
    iota = lax.broadcasted_iota(jnp.int32, s.shape, 0)
    idx = jnp.min(jnp.where(s <= m, iota, jnp.int32(2**30)), axis=0)
    idx_ref[...] = idx.reshape(idx_ref.shape)


_argmin_call = pl.pallas_call(
    _argmin_body,
    grid=(1,),
    in_specs=[
        pl.BlockSpec((_NUM_EMB, _DIM), lambda i: (0, 0)),
        pl.BlockSpec((_B, _DIM), lambda i: (0, 0)),
    ],
    out_specs=[
        pl.BlockSpec((1, 1, _B), lambda i: (0, 0, 0)),
        pl.BlockSpec((_NUM_EMB, _PAD), lambda i: (0, 0)),
    ],
    out_shape=[
        jax.ShapeDtypeStruct((1, 1, _B), jnp.int32),
        jax.ShapeDtypeStruct((_NUM_EMB, _PAD), jnp.float32),
    ],
)

_NC, _NS = 2, 16
_NW = _NC * _NS
_BPW = _B // _NW


@functools.lru_cache(maxsize=None)
def _make_gather_rows():
    mesh = plsc.VectorSubcoreMesh(core_axis_name="c", subcore_axis_name="s")

    @functools.partial(
        pl.kernel,
        mesh=mesh,
        out_type=jax.ShapeDtypeStruct((_B, _PAD), jnp.float32),
        scratch_types=[
            pltpu.VMEM((_BPW,), jnp.int32),
            pltpu.VMEM((_BPW, _PAD), jnp.float32),
            pltpu.SemaphoreType.DMA,
        ],
    )
    def _gather_rows(table_hbm, idx_hbm, out_hbm, idx_v, rows_v, sem):
        wid = lax.axis_index("s") * _NC + lax.axis_index("c")
        base = wid * _BPW
        pltpu.sync_copy(idx_hbm.at[pl.ds(base, _BPW)], idx_v)
        pltpu.async_copy(table_hbm.at[idx_v], rows_v, sem).wait()
        pltpu.sync_copy(rows_v, out_hbm.at[pl.ds(base, _BPW)])

    return _gather_rows


def kernel(x, W):
    b, s, d = x.shape
    x2 = x.reshape(b * s, d)
    idx, wp = _argmin_call(W, x2)
    out = _make_gather_rows()(wp, idx.reshape(-1))
    return out[:, :d].reshape(b, s, d)


# speedup vs baseline: 1.0238x; 1.0238x over previous
import functools

import jax
import jax.numpy as jnp
from jax import lax
from jax.experimental import pallas as pl
from jax.experimental.pallas import tpu as pltpu
from jax.experimental.pallas import tpu_sc as plsc

_NUM_EMB = 1024
_DIM = 64
_B = 1024


_BLK = 1024


def _argmin_body(w_ref, x_ref, idx_ref):
    wv = w_ref[...]
    xv = x_ref[...]
    s = lax.dot_general(
        wv, xv, (((1,), (1,)), ((), ())),
        preferred_element_type=jnp.float32,
        precision=lax.Precision.HIGHEST,
    )
    wn = 0.5 * jnp.sum(wv * wv, axis=1)
    s = wn[:, None] - s
    m = jnp.min(s, axis=0, keepdims=True)
    iota = lax.broadcasted_iota(jnp.int32, s.shape, 0)
    idx = jnp.min(jnp.where(s <= m, iota, jnp.int32(2**30)), axis=0)
    idx_ref[...] = idx.reshape(idx_ref.shape)


_argmin_call = pl.pallas_call(
    _argmin_body,
    grid=(_B // _BLK,),
    compiler_params=pltpu.CompilerParams(allow_input_fusion=[True, True]),
    in_specs=[
        pl.BlockSpec((_NUM_EMB, _DIM), lambda i: (0, 0)),
        pl.BlockSpec((_BLK, _DIM), lambda i: (i, 0)),
    ],
    out_specs=pl.BlockSpec((1, 1, _BLK), lambda i: (i, 0, 0)),
    out_shape=jax.ShapeDtypeStruct((_B // _BLK, 1, _BLK), jnp.int32),
)

_NC, _NS = 2, 16
_NW = _NC * _NS
_BPW = _B // _NW


@functools.lru_cache(maxsize=None)
def _make_gather_rows():
    mesh = plsc.VectorSubcoreMesh(core_axis_name="c", subcore_axis_name="s")

    @functools.partial(
        pl.kernel,
        mesh=mesh,
        compiler_params=pltpu.CompilerParams(use_tc_tiling_on_sc=False),
        out_type=jax.ShapeDtypeStruct((_B, _DIM), jnp.float32),
        scratch_types=[
            pltpu.VMEM((_BPW,), jnp.int32),
            pltpu.VMEM((_BPW, _DIM), jnp.float32),
            pltpu.SemaphoreType.DMA,
        ],
    )
    def _gather_rows(table_hbm, idx_hbm, out_hbm, idx_v, rows_v, sem):
        wid = lax.axis_index("s") * _NC + lax.axis_index("c")
        base = wid * _BPW
        pltpu.sync_copy(idx_hbm.at[pl.ds(base, _BPW)], idx_v)
        pltpu.async_copy(table_hbm.at[idx_v], rows_v, sem).wait()
        pltpu.sync_copy(rows_v, out_hbm.at[pl.ds(base, _BPW)])

    return _gather_rows


def kernel(x, W):
    b, s, d = x.shape
    x2 = x.reshape(b * s, d)
    idx = _argmin_call(W, x2).reshape(-1)
    out = _make_gather_rows()(W, idx)
    return out.reshape(b, s, d)
